# baseline (speedup 1.0000x reference)
import functools

import jax
import jax.numpy as jnp
import numpy as np
from jax import lax
from jax.experimental import pallas as pl
from jax.experimental.pallas import tpu as pltpu
from jax.experimental.pallas import tpu_sc as plsc

N = 50000
C = 128
K = 9
K1 = 5
K2 = K - K1

NC = 2
NS = 16
NW = NC * NS

BB = 32
CHUNKS_PW = 50
NPW = BB * CHUNKS_PW
assert NW * NPW >= N and N - NPW >= 0

_BN = 1024
NPT = 49 * _BN
assert NPT >= N + 1


def _make_mm(nk):
    def body(x_ref, w_ref, o_ref):
        i = pl.program_id(0)
        row = i * _BN + lax.broadcasted_iota(jnp.int32, (_BN, 1), 0)
        x = jnp.where(row < N, x_ref[...], 0.0)
        for k in range(nk):
            o_ref[k] = jnp.dot(x, w_ref[k], preferred_element_type=jnp.float32)

    return pl.pallas_call(
        body,
        grid=(NPT // _BN,),
        in_specs=[
            pl.BlockSpec((_BN, C), lambda i: (i, 0)),
            pl.BlockSpec((nk, C, C), lambda i: (0, 0, 0)),
        ],
        out_specs=pl.BlockSpec((nk, _BN, C), lambda i: (0, i, 0)),
        out_shape=jax.ShapeDtypeStruct((nk, NPT, C), jnp.float32),
    )


_mm_1 = _make_mm(K1)
_mm_2 = _make_mm(K2)


def _make_sc(nk, gb, first_stage):
    rows = BB * nk
    ngath = rows // gb
    assert rows % gb == 0 and gb <= 128 and rows % 16 == 0

    def koff_vec(j):
        lane = lax.iota(jnp.int32, 16)
        return ((lane + (16 * j) % nk) % nk) * NPT

    scratch = [
        pltpu.VMEM((rows,), jnp.int32),
        pltpu.VMEM((rows,), jnp.int32),
        pltpu.VMEM((ngath, gb), jnp.int32),
        pltpu.VMEM((ngath, gb), jnp.int32),
        pltpu.VMEM((rows, C), jnp.float32),
        pltpu.VMEM((rows, C), jnp.float32),
        pltpu.VMEM((BB, C), jnp.float32),
        pltpu.VMEM((BB, C), jnp.float32),
    ]
    if first_stage:
        scratch.append(pltpu.VMEM((C,), jnp.float32))
    else:
        scratch.append(pltpu.VMEM((BB, C), jnp.float32))
        scratch.append(pltpu.VMEM((BB, C), jnp.float32))
    scratch += [pltpu.SemaphoreType.DMA, pltpu.SemaphoreType.DMA]

    @functools.partial(
        pl.kernel,
        out_type=jax.ShapeDtypeStruct((N, C), jnp.float32),
        mesh=plsc.VectorSubcoreMesh(core_axis_name="c", subcore_axis_name="s"),
        scratch_types=scratch,
    )
    def sc(table_hbm, idx_hbm, aux_hbm, out_hbm, *refs):
        if first_stage:
            (idx_a, idx_b, gidx_a, gidx_b, rows_a, rows_b,
             out_a, out_b, b_v, sem_a, sem_b) = refs
            pv_a = pv_b = None
        else:
            (idx_a, idx_b, gidx_a, gidx_b, rows_a, rows_b,
             out_a, out_b, pv_a, pv_b, sem_a, sem_b) = refs

        wid = lax.axis_index("s") * NC + lax.axis_index("c")
        base = jnp.minimum(wid * NPW, N - NPW)
        if first_stage:
            pltpu.sync_copy(aux_hbm, b_v)
            bias0 = tuple(b_v[pl.ds(p * 16, 16)] for p in range(C // 16))

        def fire(c, idx_v, gidx_v, rows_v, pv_v, sem):
            nb = base + c * BB
            pltpu.sync_copy(idx_hbm.at[pl.ds(nb * nk, rows)], idx_v)
            for g in range(ngath):
                for j in range(gb // 16):
                    jj = g * (gb // 16) + j
                    v = idx_v[pl.ds(jj * 16, 16)]
                    gidx_v[g, pl.ds(j * 16, 16)] = (
                        jnp.where(v < 0, N, v) + koff_vec(jj))
            for g in range(ngath):
                pltpu.async_copy(
                    table_hbm.at[gidx_v.at[g]],
                    rows_v.at[pl.ds(g * gb, gb)],
                    sem,
                )
            if not first_stage:
                pltpu.async_copy(aux_hbm.at[pl.ds(nb, BB)], pv_v, sem)

        def process(c, gidx_v, rows_v, out_v, pv_v, sem):
            for g in range(ngath):
                pltpu.make_async_copy(
                    table_hbm.at[gidx_v.at[g]],
                    rows_v.at[pl.ds(g * gb, gb)],
                    sem,
                ).wait()
            nb = base + c * BB
            if not first_stage:
                pltpu.make_async_copy(
                    aux_hbm.at[pl.ds(nb, BB)], pv_v, sem).wait()

            def node_body(n, carry):
                r0 = n * nk
                for p in range(C // 16):
                    if first_stage:
                        acc = carry[p]
                    else:
                        acc = pv_v[n, pl.ds(p * 16, 16)]
                    for k in range(nk):
                        acc = acc + rows_v[r0 + k, pl.ds(p * 16, 16)]
                    out_v[n, pl.ds(p * 16, 16)] = acc
                return carry

            init = bias0 if first_stage else 0
            lax.fori_loop(0, BB, node_body, init, unroll=False)
            pltpu.sync_copy(out_v, out_hbm.at[pl.ds(nb, BB)])

        fire(0, idx_a, gidx_a, rows_a, pv_a, sem_a)

        def pair_body(j, carry):
            c0 = 2 * j
            fire(c0 + 1, idx_b, gidx_b, rows_b, pv_b, sem_b)
            process(c0, gidx_a, rows_a, out_a, pv_a, sem_a)

            @pl.when(j < (CHUNKS_PW // 2) - 1)
            def _():
                fire(c0 + 2, idx_a, gidx_a, rows_a, pv_a, sem_a)

            process(c0 + 1, gidx_b, rows_b, out_b, pv_b, sem_b)
            return carry

        lax.fori_loop(0, CHUNKS_PW // 2, pair_body, 0, unroll=False)

    return sc


_sc_1 = _make_sc(K1, 80, first_stage=True)
_sc_2 = _make_sc(K2, 64, first_stage=False)


def kernel(features, neigh_idx, W, b):
    Wt = W.reshape(C, K, C).transpose(1, 2, 0)
    idx = neigh_idx.astype(jnp.int32)

    t1 = _mm_1(features, Wt[:K1])
    part = _sc_1(t1.reshape(K1 * NPT, C),
                 idx[:, :K1].reshape(N * K1), b)
    t2 = _mm_2(features, Wt[K1:])
    return _sc_2(t2.reshape(K2 * NPT, C),
                 idx[:, K1:].reshape(N * K2), part)


# speedup vs baseline: 1.5428x; 1.5428x over previous
import functools

import jax
import jax.numpy as jnp
import numpy as np
from jax import lax
from jax.experimental import pallas as pl
from jax.experimental.pallas import tpu as pltpu
from jax.experimental.pallas import tpu_sc as plsc

N = 50000
C = 128
K = 9

NC = 2
NS = 16
NW = NC * NS

BB = 32
ROWS = BB * K
GB = 48
NGATH = ROWS // GB
CHUNKS_PW = 50
NPW = BB * CHUNKS_PW
assert NW * NPW >= N and N - NPW >= 0 and ((N - NPW) * K) % 8 == 0

_BN = 1024
NPT = 49 * _BN
assert NPT >= N + 1


def _mm_body(x_ref, w_ref, o_ref):
    i = pl.program_id(0)
    row = i * _BN + lax.broadcasted_iota(jnp.int32, (_BN, 1), 0)
    x = jnp.where(row < N, x_ref[...], 0.0)
    for k in range(K):
        o_ref[k] = jnp.dot(x, w_ref[k], preferred_element_type=jnp.float32)


_mm_call = pl.pallas_call(
    _mm_body,
    grid=(NPT // _BN,),
    in_specs=[
        pl.BlockSpec((_BN, C), lambda i: (i, 0)),
        pl.BlockSpec((K, C, C), lambda i: (0, 0, 0)),
    ],
    out_specs=pl.BlockSpec((K, _BN, C), lambda i: (0, i, 0)),
    out_shape=jax.ShapeDtypeStruct((K, NPT, C), jnp.float32),
)


def _koff_vec(j):
    lane = lax.iota(jnp.int32, 16)
    return ((lane + (16 * j) % K) % K) * NPT


@functools.partial(
    pl.kernel,
    out_type=jax.ShapeDtypeStruct((N, C), jnp.float32),
    mesh=plsc.VectorSubcoreMesh(core_axis_name="c", subcore_axis_name="s"),
    scratch_types=[
        pltpu.VMEM((ROWS,), jnp.int32),
        pltpu.VMEM((ROWS,), jnp.int32),
        pltpu.VMEM((NGATH, GB), jnp.int32),
        pltpu.VMEM((NGATH, GB), jnp.int32),
        pltpu.VMEM((ROWS, C), jnp.float32),
        pltpu.VMEM((ROWS, C), jnp.float32),
        pltpu.VMEM((BB, C), jnp.float32),
        pltpu.VMEM((BB, C), jnp.float32),
        pltpu.VMEM((C,), jnp.float32),
        pltpu.SemaphoreType.DMA,
        pltpu.SemaphoreType.DMA,
    ],
)
def _sc_gather(table_hbm, idx_hbm, b_hbm, out_hbm,
               idx_a, idx_b, gidx_a, gidx_b, rows_a, rows_b,
               out_a, out_b, b_v, sem_a, sem_b):
    wid = lax.axis_index("s") * NC + lax.axis_index("c")
    base = jnp.minimum(wid * NPW, N - NPW)
    pltpu.sync_copy(b_hbm, b_v)
    bias0 = tuple(b_v[pl.ds(p * 16, 16)] for p in range(C // 16))

    def fire(c, idx_v, gidx_v, rows_v, sem):
        fb = (base + c * BB) * K
        pltpu.sync_copy(idx_hbm.at[pl.ds(fb, ROWS)], idx_v)
        for g in range(NGATH):
            for j in range(GB // 16):
                jj = g * (GB // 16) + j
                v = idx_v[pl.ds(jj * 16, 16)]
                gidx_v[g, pl.ds(j * 16, 16)] = (
                    jnp.where(v < 0, N, v) + _koff_vec(jj))
        for g in range(NGATH):
            pltpu.async_copy(
                table_hbm.at[gidx_v.at[g]],
                rows_v.at[pl.ds(g * GB, GB)],
                sem,
            )

    def process(c, gidx_v, rows_v, out_v, sem):
        for g in range(NGATH):
            pltpu.make_async_copy(
                table_hbm.at[gidx_v.at[g]],
                rows_v.at[pl.ds(g * GB, GB)],
                sem,
            ).wait()

        def node_body(n, bias):
            r0 = n * K
            for p in range(C // 16):
                acc = bias[p]
                for k in range(K):
                    acc = acc + rows_v[r0 + k, pl.ds(p * 16, 16)]
                out_v[n, pl.ds(p * 16, 16)] = acc
            return bias

        lax.fori_loop(0, BB, node_body, bias0, unroll=False)
        nb = base + c * BB
        pltpu.sync_copy(out_v, out_hbm.at[pl.ds(nb, BB)])

    fire(0, idx_a, gidx_a, rows_a, sem_a)

    def pair_body(j, carry):
        c0 = 2 * j
        fire(c0 + 1, idx_b, gidx_b, rows_b, sem_b)
        process(c0, gidx_a, rows_a, out_a, sem_a)

        @pl.when(j < (CHUNKS_PW // 2) - 1)
        def _():
            fire(c0 + 2, idx_a, gidx_a, rows_a, sem_a)

        process(c0 + 1, gidx_b, rows_b, out_b, sem_b)
        return carry

    lax.fori_loop(0, CHUNKS_PW // 2, pair_body, 0, unroll=False)


def kernel(features, neigh_idx, W, b):
    Wt = W.reshape(C, K, C).transpose(1, 2, 0)
    tables = _mm_call(features, Wt)
    merged = tables.reshape(K * NPT, C)
    idx_flat = neigh_idx.reshape(N * K).astype(jnp.int32)
    return _sc_gather(merged, idx_flat, b)
